# baseline (speedup 1.0000x reference)
import functools

import jax
import jax.numpy as jnp
from jax import lax
from jax.experimental import pallas as pl
from jax.experimental.pallas import tpu as pltpu
from jax.experimental.pallas import tpu_sc as plsc

OW = OL = OH = 4
S = 2
COLS = OW * OL * OH * S * S * S * 8
NBINS = OW * OL * OH
PAIRS_PER_ROW = S * S * S * 8

NC, NS = 2, 16
NW = NC * NS


def _phase1_call(rois_s, W, L, H):
    N = rois_s.shape[0]
    BLK = 64
    Wf, Lf, Hf = float(W), float(L), float(H)

    def body(r_ref, idx_ref, wgt_ref):
        r = r_ref[...]
        b = r[:, 0:1].astype(jnp.int32)
        cx, cy, cz = r[:, 1:2], r[:, 2:3], r[:, 3:4]
        rw = jnp.maximum(r[:, 4:5], 1e-3)
        rl = jnp.maximum(r[:, 5:6], 1e-3)
        rh = jnp.maximum(r[:, 6:7], 1e-3)
        th = r[:, 7:8]
        cth, sth = jnp.cos(th), jnp.sin(th)

        m = lax.broadcasted_iota(jnp.int32, (BLK, COLS), 1)
        p = (m >> 10) * 2 + ((m >> 5) & 1)
        q = ((m >> 8) & 3) * 2 + ((m >> 4) & 1)
        t = ((m >> 6) & 3) * 2 + ((m >> 3) & 1)
        cxb = (m & 1).astype(jnp.float32)
        cyb = ((m >> 1) & 1).astype(jnp.float32)
        czb = ((m >> 2) & 1).astype(jnp.float32)
        pf = p.astype(jnp.float32) * 0.5 + 0.25
        qf = q.astype(jnp.float32) * 0.5 + 0.25
        tf = t.astype(jnp.float32) * 0.5 + 0.25

        xl = pf * (rw * (1.0 / OW)) - rw * 0.5
        yl = qf * (rl * (1.0 / OL)) - rl * 0.5
        zl = tf * (rh * (1.0 / OH)) - rh * 0.5
        px = cx + xl * cth - yl * sth
        py = cy + xl * sth + yl * cth
        pz = cz + zl
        valid = ((px > -1.0) & (px < Wf) & (py > -1.0) & (py < Lf)
                 & (pz > -1.0) & (pz < Hf))
        x = jnp.clip(px, 0.0, Wf - 1.0)
        y = jnp.clip(py, 0.0, Lf - 1.0)
        z = jnp.clip(pz, 0.0, Hf - 1.0)
        x0 = jnp.floor(x)
        y0 = jnp.floor(y)
        z0 = jnp.floor(z)
        lx, ly, lz = x - x0, y - y0, z - z0
        xi = jnp.minimum(x0 + cxb, Wf - 1.0).astype(jnp.int32)
        yi = jnp.minimum(y0 + cyb, Lf - 1.0).astype(jnp.int32)
        zi = jnp.minimum(z0 + czb, Hf - 1.0).astype(jnp.int32)
        wx = cxb * lx + (1.0 - cxb) * (1.0 - lx)
        wy = cyb * ly + (1.0 - cyb) * (1.0 - ly)
        wz = czb * lz + (1.0 - czb) * (1.0 - lz)
        idx_ref[...] = ((b * W + xi) * L + yi) * H + zi
        wgt_ref[...] = wx * wy * wz * valid.astype(jnp.float32) * 0.125

    return pl.pallas_call(
        body,
        grid=(N // BLK,),
        in_specs=[pl.BlockSpec((BLK, 8), lambda i: (i, 0))],
        out_specs=[pl.BlockSpec((BLK, COLS), lambda i: (i, 0)),
                   pl.BlockSpec((BLK, COLS), lambda i: (i, 0))],
        out_shape=[jax.ShapeDtypeStruct((N, COLS), jnp.int32),
                   jax.ShapeDtypeStruct((N, COLS), jnp.float32)],
    )(rois_s)


def _phase2_call(table, idxf, wgtf, n_rows):
    C = table.shape[1]
    NBUF = 4
    RPW = n_rows // NW
    NSUP = 8
    SROWS = RPW // NSUP
    PAIRS = SROWS * PAIRS_PER_ROW
    G = 128
    NSTEP = PAIRS // G
    mesh = plsc.VectorSubcoreMesh(core_axis_name="c", subcore_axis_name="s",
                                  num_cores=NC, num_subcores=NS)

    @functools.partial(
        pl.kernel,
        out_type=jax.ShapeDtypeStruct((n_rows, C), jnp.float32),
        mesh=mesh,
        scratch_types=[
            pltpu.VMEM((PAIRS,), jnp.int32),
            pltpu.VMEM((PAIRS,), jnp.float32),
            pltpu.VMEM((NBUF, G, C), jnp.float32),
            pltpu.VMEM((SROWS, C), jnp.float32),
        ] + [pltpu.SemaphoreType.DMA] * NBUF,
        compiler_params=pltpu.CompilerParams(use_tc_tiling_on_sc=False),
    )
    def run(table_h, idx_h, wgt_h, out_h, idx_v, wgt_v, gbuf, obuf, *sems):
        wid = lax.axis_index("s") * NC + lax.axis_index("c")

        def gdesc(st, bi):
            return pltpu.make_async_copy(
                table_h.at[idx_v.at[pl.ds(st * G, G)]], gbuf.at[bi], sems[bi])

        def superstep(sup, carry):
            off = wid * (RPW * PAIRS_PER_ROW) + sup * PAIRS
            pltpu.sync_copy(idx_h.at[pl.ds(off, PAIRS)], idx_v)
            pltpu.sync_copy(wgt_h.at[pl.ds(off, PAIRS)], wgt_v)
            for bi in range(NBUF):
                gdesc(bi, bi).start()

            def steps(i2, c2):
                for bi in range(NBUF):
                    st = i2 * NBUF + bi
                    gdesc(st, bi).wait()
                    for half in range(2):
                        wbase = st * G + half * 64
                        rbase = half * 64

                        def grp(g, acc):
                            wvec = wgt_v[pl.ds(wbase + g * 16, 16)]
                            a0, a1, a2, a3 = acc
                            for lane in range(16):
                                w = wvec[lane]
                                rr = rbase + g * 16 + lane
                                a0 = a0 + w * gbuf[bi, rr, pl.ds(0, 16)]
                                a1 = a1 + w * gbuf[bi, rr, pl.ds(16, 16)]
                                a2 = a2 + w * gbuf[bi, rr, pl.ds(32, 16)]
                                a3 = a3 + w * gbuf[bi, rr, pl.ds(48, 16)]
                            return (a0, a1, a2, a3)

                        z16 = jnp.zeros((16,), jnp.float32)
                        a0, a1, a2, a3 = lax.fori_loop(
                            0, 4, grp, (z16, z16, z16, z16))
                        orow = st * 2 + half
                        obuf[orow, pl.ds(0, 16)] = a0
                        obuf[orow, pl.ds(16, 16)] = a1
                        obuf[orow, pl.ds(32, 16)] = a2
                        obuf[orow, pl.ds(48, 16)] = a3

                    @pl.when(st + NBUF < NSTEP)
                    def _():
                        gdesc(st + NBUF, bi).start()
                return c2

            lax.fori_loop(0, NSTEP // NBUF, steps, 0)
            pltpu.sync_copy(obuf,
                            out_h.at[pl.ds(wid * RPW + sup * SROWS, SROWS)])
            return carry

        lax.fori_loop(0, NSUP, superstep, 0)

    return run(table, idxf, wgtf)


def kernel(input, rois, spatial_scale):
    B, C, W, L, H = input.shape
    N = rois.shape[0]
    ss = jnp.asarray(spatial_scale, jnp.float32)
    scale = jnp.concatenate(
        [jnp.ones((1,), jnp.float32), jnp.full((6,), ss, jnp.float32),
         jnp.ones((1,), jnp.float32)])
    rois_s = rois * scale[None, :]

    idx, wgt = _phase1_call(rois_s, W, L, H)

    tc2 = jnp.transpose(input, (0, 2, 3, 4, 1)).reshape(B * W * L * H // 2,
                                                        2 * C)
    tc2 = lax.optimization_barrier(tc2)
    table = tc2.reshape(B * W * L * H, C)
    n_rows = N * NBINS
    out = _phase2_call(table, idx.reshape(-1), wgt.reshape(-1), n_rows)
    return jnp.transpose(out.reshape(N, OW, OL, OH, C), (0, 4, 1, 2, 3))


# speedup vs baseline: 1.0146x; 1.0146x over previous
import functools

import jax
import jax.numpy as jnp
from jax import lax
from jax.experimental import pallas as pl
from jax.experimental.pallas import tpu as pltpu
from jax.experimental.pallas import tpu_sc as plsc

OW = OL = OH = 4
S = 2
COLS = OW * OL * OH * S * S * S * 8
NBINS = OW * OL * OH
PAIRS_PER_ROW = S * S * S * 8

NC, NS = 2, 16
NW = NC * NS


def _phase1_call(rois_s, W, L, H):
    N = rois_s.shape[0]
    BLK = 64
    Wf, Lf, Hf = float(W), float(L), float(H)

    def body(r_ref, idx_ref, wgt_ref):
        r = r_ref[...]
        b = r[:, 0:1].astype(jnp.int32)
        cx, cy, cz = r[:, 1:2], r[:, 2:3], r[:, 3:4]
        rw = jnp.maximum(r[:, 4:5], 1e-3)
        rl = jnp.maximum(r[:, 5:6], 1e-3)
        rh = jnp.maximum(r[:, 6:7], 1e-3)
        th = r[:, 7:8]
        cth, sth = jnp.cos(th), jnp.sin(th)

        m = lax.broadcasted_iota(jnp.int32, (BLK, COLS), 1)
        p = (m >> 10) * 2 + ((m >> 5) & 1)
        q = ((m >> 8) & 3) * 2 + ((m >> 4) & 1)
        t = ((m >> 6) & 3) * 2 + ((m >> 3) & 1)
        cxb = (m & 1).astype(jnp.float32)
        cyb = ((m >> 1) & 1).astype(jnp.float32)
        czb = ((m >> 2) & 1).astype(jnp.float32)
        pf = p.astype(jnp.float32) * 0.5 + 0.25
        qf = q.astype(jnp.float32) * 0.5 + 0.25
        tf = t.astype(jnp.float32) * 0.5 + 0.25

        xl = pf * (rw * (1.0 / OW)) - rw * 0.5
        yl = qf * (rl * (1.0 / OL)) - rl * 0.5
        zl = tf * (rh * (1.0 / OH)) - rh * 0.5
        px = cx + xl * cth - yl * sth
        py = cy + xl * sth + yl * cth
        pz = cz + zl
        valid = ((px > -1.0) & (px < Wf) & (py > -1.0) & (py < Lf)
                 & (pz > -1.0) & (pz < Hf))
        x = jnp.clip(px, 0.0, Wf - 1.0)
        y = jnp.clip(py, 0.0, Lf - 1.0)
        z = jnp.clip(pz, 0.0, Hf - 1.0)
        x0 = jnp.floor(x)
        y0 = jnp.floor(y)
        z0 = jnp.floor(z)
        lx, ly, lz = x - x0, y - y0, z - z0
        xi = jnp.minimum(x0 + cxb, Wf - 1.0).astype(jnp.int32)
        yi = jnp.minimum(y0 + cyb, Lf - 1.0).astype(jnp.int32)
        zi = jnp.minimum(z0 + czb, Hf - 1.0).astype(jnp.int32)
        wx = cxb * lx + (1.0 - cxb) * (1.0 - lx)
        wy = cyb * ly + (1.0 - cyb) * (1.0 - ly)
        wz = czb * lz + (1.0 - czb) * (1.0 - lz)
        idx_ref[...] = ((b * W + xi) * L + yi) * H + zi
        wgt_ref[...] = wx * wy * wz * valid.astype(jnp.float32) * 0.125

    return pl.pallas_call(
        body,
        grid=(N // BLK,),
        in_specs=[pl.BlockSpec((BLK, 8), lambda i: (i, 0))],
        out_specs=[pl.BlockSpec((BLK, COLS), lambda i: (i, 0)),
                   pl.BlockSpec((BLK, COLS), lambda i: (i, 0))],
        out_shape=[jax.ShapeDtypeStruct((N, COLS), jnp.int32),
                   jax.ShapeDtypeStruct((N, COLS), jnp.float32)],
    )(rois_s)


def _phase2_call(table, idxf, wgtf, n_rows):
    C = table.shape[1]
    NBUF = 4
    RPW = n_rows // NW
    NSUP = 4
    SROWS = RPW // NSUP
    PAIRS = SROWS * PAIRS_PER_ROW
    G = 128
    NSTEP = PAIRS // G
    mesh = plsc.VectorSubcoreMesh(core_axis_name="c", subcore_axis_name="s",
                                  num_cores=NC, num_subcores=NS)

    @functools.partial(
        pl.kernel,
        out_type=jax.ShapeDtypeStruct((n_rows, C), jnp.float32),
        mesh=mesh,
        scratch_types=[
            pltpu.VMEM((PAIRS,), jnp.int32),
            pltpu.VMEM((PAIRS,), jnp.float32),
            pltpu.VMEM((NBUF, G, C), jnp.float32),
            pltpu.VMEM((SROWS, C), jnp.float32),
        ] + [pltpu.SemaphoreType.DMA] * NBUF,
        compiler_params=pltpu.CompilerParams(use_tc_tiling_on_sc=False),
    )
    def run(table_h, idx_h, wgt_h, out_h, idx_v, wgt_v, gbuf, obuf, *sems):
        wid = lax.axis_index("s") * NC + lax.axis_index("c")

        def gdesc(st, bi):
            return pltpu.make_async_copy(
                table_h.at[idx_v.at[pl.ds(st * G, G)]], gbuf.at[bi], sems[bi])

        def superstep(sup, carry):
            off = wid * (RPW * PAIRS_PER_ROW) + sup * PAIRS
            pltpu.sync_copy(idx_h.at[pl.ds(off, PAIRS)], idx_v)
            pltpu.sync_copy(wgt_h.at[pl.ds(off, PAIRS)], wgt_v)
            for bi in range(NBUF):
                gdesc(bi, bi).start()

            def steps(i2, c2):
                for bi in range(NBUF):
                    st = i2 * NBUF + bi
                    gdesc(st, bi).wait()
                    for half in range(2):
                        wbase = st * G + half * 64
                        rbase = half * 64

                        def grp(g, acc):
                            wvec = wgt_v[pl.ds(wbase + g * 16, 16)]
                            a0, a1, a2, a3 = acc
                            for lane in range(16):
                                w = wvec[lane]
                                rr = rbase + g * 16 + lane
                                a0 = a0 + w * gbuf[bi, rr, pl.ds(0, 16)]
                                a1 = a1 + w * gbuf[bi, rr, pl.ds(16, 16)]
                                a2 = a2 + w * gbuf[bi, rr, pl.ds(32, 16)]
                                a3 = a3 + w * gbuf[bi, rr, pl.ds(48, 16)]
                            return (a0, a1, a2, a3)

                        z16 = jnp.zeros((16,), jnp.float32)
                        a0, a1, a2, a3 = lax.fori_loop(
                            0, 4, grp, (z16, z16, z16, z16))
                        orow = st * 2 + half
                        obuf[orow, pl.ds(0, 16)] = a0
                        obuf[orow, pl.ds(16, 16)] = a1
                        obuf[orow, pl.ds(32, 16)] = a2
                        obuf[orow, pl.ds(48, 16)] = a3

                    @pl.when(st + NBUF < NSTEP)
                    def _():
                        gdesc(st + NBUF, bi).start()
                return c2

            lax.fori_loop(0, NSTEP // NBUF, steps, 0)
            pltpu.sync_copy(obuf,
                            out_h.at[pl.ds(wid * RPW + sup * SROWS, SROWS)])
            return carry

        lax.fori_loop(0, NSUP, superstep, 0)

    return run(table, idxf, wgtf)


def kernel(input, rois, spatial_scale):
    B, C, W, L, H = input.shape
    N = rois.shape[0]
    ss = jnp.asarray(spatial_scale, jnp.float32)
    scale = jnp.concatenate(
        [jnp.ones((1,), jnp.float32), jnp.full((6,), ss, jnp.float32),
         jnp.ones((1,), jnp.float32)])
    rois_s = rois * scale[None, :]

    idx, wgt = _phase1_call(rois_s, W, L, H)

    table = jnp.transpose(input, (0, 2, 3, 4, 1)).reshape(B * W * L * H, C)
    n_rows = N * NBINS
    out = _phase2_call(table, idx.reshape(-1), wgt.reshape(-1), n_rows)
    return jnp.transpose(out.reshape(N, OW, OL, OH, C), (0, 4, 1, 2, 3))
